# baseline (speedup 1.0000x reference)
import functools

import jax
import jax.numpy as jnp
from jax import lax
from jax.experimental import pallas as pl
from jax.experimental.pallas import tpu as pltpu
from jax.experimental.pallas import tpu_sc as plsc

NC = 2
NS = 16
CHUNK = 128
D = 64


def _round_up(v, m):
    return (v + m - 1) // m * m


K = 4

NCH0 = 160
NCH1 = 0


def _make_segsum(n_nodes, n_edges_pad):
    assert n_edges_pad == NS * (NCH0 + NCH1) * CHUNK
    rows_pad = _round_up(n_nodes + 1, NS * 64)
    zrows = rows_pad // NS
    mesh = plsc.VectorSubcoreMesh(core_axis_name="c", subcore_axis_name="s")

    @functools.partial(
        pl.kernel,
        mesh=mesh,
        out_type=jax.ShapeDtypeStruct((NC * rows_pad, D), jnp.float32),
        scratch_types=[
            pltpu.VMEM((K, CHUNK, D), jnp.float32),
            pltpu.VMEM((64, D), jnp.float32),
            pltpu.VMEM_SHARED((rows_pad, D), jnp.float32),
            pltpu.SemaphoreType.DMA,
            pltpu.SemaphoreType.DMA,
            pltpu.SemaphoreType.DMA,
        ] + [pltpu.VMEM((CHUNK,), jnp.int32)] * (4 * K),
        compiler_params=pltpu.CompilerParams(use_tc_tiling_on_sc=False),
    )
    def segsum(table_hbm, src_hbm, dst_hbm, out_hbm, rows, zbuf,
               acc, gsem, ssem, isem, *idxrefs):
        didx = idxrefs[:2 * K]
        sidx = idxrefs[2 * K:]
        cid = lax.axis_index("c")
        sid = lax.axis_index("s")
        is0 = cid == 0
        crow0 = jnp.where(is0, sid * NCH0, NS * NCH0 + sid * NCH1)
        npairs = jnp.where(is0, NCH0 // (2 * K), NCH1 // (2 * K))
        last = jnp.where(is0, NCH0 // K, NCH1 // K) - 1

        nrows_total = NS * (NCH0 + NCH1)

        def idx_start(g, bank):
            return [pltpu.async_copy(
                        h.at[pl.ds(jnp.minimum(crow0 + g * K + b,
                                               nrows_total - 1) * CHUNK,
                                   CHUNK)],
                        r[bank * K + b], isem)
                    for b in range(K) for h, r in ((src_hbm, sidx),
                                                   (dst_hbm, didx))]

        di0 = idx_start(0, 0)

        zero16 = jnp.zeros((16,), jnp.float32)

        def zrow(i, _):
            for k4 in range(D // 16):
                zbuf[i, pl.ds(k4 * 16, 16)] = zero16
            return 0

        lax.fori_loop(0, 64, zrow, 0)
        dz = [pltpu.async_copy(zbuf,
                               acc.at[pl.ds(sid * zrows + j * 64, 64)], ssem)
              for j in range(zrows // 64)]
        for d in dz:
            d.wait()
        for d in di0:
            d.wait()
        plsc.subcore_barrier()

        def run_group(g, bank, gnext, nbank):
            di = idx_start(gnext, nbank)
            dg = [pltpu.async_copy(table_hbm.at[sidx[bank * K + b]],
                                   rows.at[b], gsem) for b in range(K)]
            for d in dg:
                d.wait()
            ds = [pltpu.async_copy(rows.at[b], acc.at[didx[bank * K + b]],
                                   ssem, add=True) for b in range(K)]
            for d in ds:
                d.wait()
            for d in di:
                d.wait()

        def pair(pp, _):
            g0 = 2 * pp
            g1 = g0 + 1
            run_group(g0, 0, jnp.minimum(g1, last), 1)
            run_group(g1, 1, jnp.minimum(g1 + 1, last), 0)
            return 0

        lax.fori_loop(0, npairs, pair, 0)
        plsc.subcore_barrier()

        r0 = sid * zrows
        pltpu.sync_copy(acc.at[pl.ds(r0, zrows)],
                        out_hbm.at[pl.ds(cid * rows_pad + r0, zrows)])

    return segsum, rows_pad


_BLK = 1000


def _proj(x, w):
    n, d_in = x.shape

    def body(x_ref, w_ref, o_ref):
        o_ref[...] = jnp.dot(x_ref[...], w_ref[...],
                             preferred_element_type=jnp.float32)

    return pl.pallas_call(
        body,
        grid=(n // _BLK,),
        in_specs=[pl.BlockSpec((_BLK, d_in), lambda i: (i, 0)),
                  pl.BlockSpec((d_in, D), lambda i: (0, 0))],
        out_specs=pl.BlockSpec((_BLK, D), lambda i: (i, 0)),
        out_shape=jax.ShapeDtypeStruct((n, D), jnp.float32),
    )(x, w)


def _mlp_stage(p, a0, a1, ba, wb, bb, wn):
    n = p.shape[0]
    nb = n // _BLK
    last = wn is None

    def body(p_ref, a0_ref, a1_ref, ba_ref, wb_ref, bb_ref, *rest):
        o_ref = rest[-1]
        s = p_ref[...] + a0_ref[...] + a1_ref[...] + ba_ref[...]
        h = jnp.dot(jnp.maximum(s, 0.0), wb_ref[...],
                    preferred_element_type=jnp.float32) + bb_ref[...]
        h = jnp.where(h > 0, h, 0.01 * h)
        if not last:
            h = jnp.dot(h, rest[0][...], preferred_element_type=jnp.float32)
        o_ref[...] = h

    in_specs = [
        pl.BlockSpec((_BLK, D), lambda i: (i, 0)),
        pl.BlockSpec((_BLK, D), lambda i: (i, 0)),
        pl.BlockSpec((_BLK, D), lambda i: (i, 0)),
        pl.BlockSpec((1, D), lambda i: (0, 0)),
        pl.BlockSpec((D, D), lambda i: (0, 0)),
        pl.BlockSpec((1, D), lambda i: (0, 0)),
    ]
    args = [p, a0, a1, ba, wb, bb]
    if not last:
        in_specs.append(pl.BlockSpec((D, D), lambda i: (0, 0)))
        args.append(wn)

    return pl.pallas_call(
        body,
        grid=(nb,),
        in_specs=in_specs,
        out_specs=pl.BlockSpec((_BLK, D), lambda i: (i, 0)),
        out_shape=jax.ShapeDtypeStruct((n, D), jnp.float32),
    )(*args)


def kernel(x, edge_index, w1a, b1a, w1b, b1b, w2a, b2a, w2b, b2b):
    n_nodes = x.shape[0]
    n_edges = edge_index.shape[1]
    e_pad = NS * (NCH0 + NCH1) * CHUNK
    assert n_edges <= e_pad

    src = edge_index[0]
    dst = edge_index[1]
    npad = e_pad - n_edges
    if npad:
        src = jnp.concatenate([src, jnp.zeros((npad,), jnp.int32)])
        dst = jnp.concatenate([dst, jnp.full((npad,), n_nodes, jnp.int32)])

    segsum, rows_pad = _make_segsum(n_nodes, e_pad)

    def split(parts):
        return (parts[:n_nodes], parts[rows_pad:rows_pad + n_nodes])

    b1a2, b1b2, b2a2, b2b2 = (b.reshape(1, D) for b in (b1a, b1b, b2a, b2b))

    p = _proj(x, w1a)
    a0, a1 = split(segsum(p, src, dst))
    q = _mlp_stage(p, a0, a1, b1a2, w1b, b1b2, w2a)
    qa0, qa1 = split(segsum(q, src, dst))
    return _mlp_stage(q, qa0, qa1, b2a2, w2b, b2b2, None)


# speedup vs baseline: 1.4752x; 1.4752x over previous
import functools

import jax
import jax.numpy as jnp
from jax import lax
from jax.experimental import pallas as pl
from jax.experimental.pallas import tpu as pltpu
from jax.experimental.pallas import tpu_sc as plsc

NC = 2
NS = 16
CHUNK = 128
D = 64


def _round_up(v, m):
    return (v + m - 1) // m * m


K = 4

NCH0 = 144
NCH1 = 16


def _make_segsum(n_nodes, n_edges_pad):
    assert n_edges_pad == NS * (NCH0 + NCH1) * CHUNK
    rows_pad = _round_up(n_nodes + 1, NS * 64)
    zrows = rows_pad // NS
    mesh = plsc.VectorSubcoreMesh(core_axis_name="c", subcore_axis_name="s")

    @functools.partial(
        pl.kernel,
        mesh=mesh,
        out_type=jax.ShapeDtypeStruct((NC * rows_pad, D), jnp.float32),
        scratch_types=[
            pltpu.VMEM((K, CHUNK, D), jnp.float32),
            pltpu.VMEM((64, D), jnp.float32),
            pltpu.VMEM_SHARED((rows_pad, D), jnp.float32),
            pltpu.SemaphoreType.DMA,
            pltpu.SemaphoreType.DMA,
            pltpu.SemaphoreType.DMA,
        ] + [pltpu.VMEM((CHUNK,), jnp.int32)] * (4 * K),
        compiler_params=pltpu.CompilerParams(use_tc_tiling_on_sc=False),
    )
    def segsum(table_hbm, src_hbm, dst_hbm, out_hbm, rows, zbuf,
               acc, gsem, ssem, isem, *idxrefs):
        didx = idxrefs[:2 * K]
        sidx = idxrefs[2 * K:]
        cid = lax.axis_index("c")
        sid = lax.axis_index("s")
        is0 = cid == 0
        crow0 = jnp.where(is0, sid * NCH0, NS * NCH0 + sid * NCH1)
        npairs = jnp.where(is0, NCH0 // (2 * K), NCH1 // (2 * K))
        last = jnp.where(is0, NCH0 // K, NCH1 // K) - 1

        nrows_total = NS * (NCH0 + NCH1)

        def idx_start(g, bank):
            return [pltpu.async_copy(
                        h.at[pl.ds(jnp.minimum(crow0 + g * K + b,
                                               nrows_total - 1) * CHUNK,
                                   CHUNK)],
                        r[bank * K + b], isem)
                    for b in range(K) for h, r in ((src_hbm, sidx),
                                                   (dst_hbm, didx))]

        di0 = idx_start(0, 0)

        zero16 = jnp.zeros((16,), jnp.float32)

        def zrow(i, _):
            for k4 in range(D // 16):
                zbuf[i, pl.ds(k4 * 16, 16)] = zero16
            return 0

        lax.fori_loop(0, 64, zrow, 0)
        dz = [pltpu.async_copy(zbuf,
                               acc.at[pl.ds(sid * zrows + j * 64, 64)], ssem)
              for j in range(zrows // 64)]
        for d in dz:
            d.wait()
        for d in di0:
            d.wait()
        plsc.subcore_barrier()

        def run_group(g, bank, gnext, nbank):
            di = idx_start(gnext, nbank)
            dg = [pltpu.async_copy(table_hbm.at[sidx[bank * K + b]],
                                   rows.at[b], gsem) for b in range(K)]
            for d in dg:
                d.wait()
            ds = [pltpu.async_copy(rows.at[b], acc.at[didx[bank * K + b]],
                                   ssem, add=True) for b in range(K)]
            for d in ds:
                d.wait()
            for d in di:
                d.wait()

        def pair(pp, _):
            g0 = 2 * pp
            g1 = g0 + 1
            run_group(g0, 0, jnp.minimum(g1, last), 1)
            run_group(g1, 1, jnp.minimum(g1 + 1, last), 0)
            return 0

        lax.fori_loop(0, npairs, pair, 0)
        plsc.subcore_barrier()

        r0 = sid * zrows
        pltpu.sync_copy(acc.at[pl.ds(r0, zrows)],
                        out_hbm.at[pl.ds(cid * rows_pad + r0, zrows)])

    return segsum, rows_pad


_BLK = 1000


def _proj(x, w):
    n, d_in = x.shape

    def body(x_ref, w_ref, o_ref):
        o_ref[...] = jnp.dot(x_ref[...], w_ref[...],
                             preferred_element_type=jnp.float32)

    return pl.pallas_call(
        body,
        grid=(n // _BLK,),
        in_specs=[pl.BlockSpec((_BLK, d_in), lambda i: (i, 0)),
                  pl.BlockSpec((d_in, D), lambda i: (0, 0))],
        out_specs=pl.BlockSpec((_BLK, D), lambda i: (i, 0)),
        out_shape=jax.ShapeDtypeStruct((n, D), jnp.float32),
    )(x, w)


def _mlp_stage(p, a0, a1, ba, wb, bb, wn):
    n = p.shape[0]
    nb = n // _BLK
    last = wn is None

    def body(p_ref, a0_ref, a1_ref, ba_ref, wb_ref, bb_ref, *rest):
        o_ref = rest[-1]
        s = p_ref[...] + a0_ref[...] + a1_ref[...] + ba_ref[...]
        h = jnp.dot(jnp.maximum(s, 0.0), wb_ref[...],
                    preferred_element_type=jnp.float32) + bb_ref[...]
        h = jnp.where(h > 0, h, 0.01 * h)
        if not last:
            h = jnp.dot(h, rest[0][...], preferred_element_type=jnp.float32)
        o_ref[...] = h

    in_specs = [
        pl.BlockSpec((_BLK, D), lambda i: (i, 0)),
        pl.BlockSpec((_BLK, D), lambda i: (i, 0)),
        pl.BlockSpec((_BLK, D), lambda i: (i, 0)),
        pl.BlockSpec((1, D), lambda i: (0, 0)),
        pl.BlockSpec((D, D), lambda i: (0, 0)),
        pl.BlockSpec((1, D), lambda i: (0, 0)),
    ]
    args = [p, a0, a1, ba, wb, bb]
    if not last:
        in_specs.append(pl.BlockSpec((D, D), lambda i: (0, 0)))
        args.append(wn)

    return pl.pallas_call(
        body,
        grid=(nb,),
        in_specs=in_specs,
        out_specs=pl.BlockSpec((_BLK, D), lambda i: (i, 0)),
        out_shape=jax.ShapeDtypeStruct((n, D), jnp.float32),
    )(*args)


def kernel(x, edge_index, w1a, b1a, w1b, b1b, w2a, b2a, w2b, b2b):
    n_nodes = x.shape[0]
    n_edges = edge_index.shape[1]
    e_pad = NS * (NCH0 + NCH1) * CHUNK
    assert n_edges <= e_pad

    src = edge_index[0]
    dst = edge_index[1]
    npad = e_pad - n_edges
    if npad:
        src = jnp.concatenate([src, jnp.zeros((npad,), jnp.int32)])
        dst = jnp.concatenate([dst, jnp.full((npad,), n_nodes, jnp.int32)])

    segsum, rows_pad = _make_segsum(n_nodes, e_pad)

    def split(parts):
        return (parts[:n_nodes], parts[rows_pad:rows_pad + n_nodes])

    b1a2, b1b2, b2a2, b2b2 = (b.reshape(1, D) for b in (b1a, b1b, b2a, b2b))

    p = _proj(x, w1a)
    a0, a1 = split(segsum(p, src, dst))
    q = _mlp_stage(p, a0, a1, b1a2, w1b, b1b2, w2a)
    qa0, qa1 = split(segsum(q, src, dst))
    return _mlp_stage(q, qa0, qa1, b2a2, w2b, b2b2, None)
